# baseline (speedup 1.0000x reference)
import functools

import numpy as np
import jax
import jax.numpy as jnp
from jax import lax
from jax.experimental import pallas as pl
from jax.experimental.pallas import tpu as pltpu
from jax.experimental.pallas import tpu_sc as plsc

_NC, _NS = 2, 16
_NW = _NC * _NS
_CH = 128
_K = 4


def _sc_gather(table, idx2d):
    n_ch_total, ch = idx2d.shape
    v, d = table.shape
    assert ch == _CH
    n_rows = n_ch_total * _CH
    n_ch = n_ch_total // _NW
    n_g = n_ch // _K
    assert n_ch_total % _NW == 0 and n_ch % _K == 0 and n_g % 2 == 0

    mesh = plsc.VectorSubcoreMesh(
        core_axis_name="c", subcore_axis_name="s",
        num_cores=_NC, num_subcores=_NS)

    @functools.partial(
        pl.kernel,
        out_type=jax.ShapeDtypeStruct((n_rows, d), jnp.float32),
        mesh=mesh,
        scratch_types=[
            pltpu.VMEM((n_ch, _CH), jnp.int32),
            pltpu.VMEM((2, _K * _CH, d), jnp.float32),
            pltpu.SemaphoreType.DMA,
            pltpu.SemaphoreType.DMA,
        ],
        compiler_params=pltpu.CompilerParams(use_tc_tiling_on_sc=False),
    )
    def gather_kernel(table_hbm, idx_hbm, out_hbm, idx_v, rows_v, sem0, sem1):
        wid = lax.axis_index("s") * _NC + lax.axis_index("c")
        ch0 = wid * n_ch
        sems = (sem0, sem1)

        pltpu.sync_copy(idx_hbm.at[pl.ds(ch0, n_ch)], idx_v)

        def issue_group(g, slot):
            for k in range(_K):
                pltpu.async_copy(
                    table_hbm.at[idx_v.at[g * _K + k]],
                    rows_v.at[slot].at[pl.ds(k * _CH, _CH)],
                    sems[slot])

        def drain_group(slot):
            for k in range(_K):
                pltpu.make_async_copy(
                    table_hbm.at[idx_v.at[0]],
                    rows_v.at[slot].at[pl.ds(k * _CH, _CH)],
                    sems[slot]).wait()

        issue_group(0, 0)

        @pl.loop(0, n_g // 2)
        def _(q):
            for s in range(2):
                g = q * 2 + s
                gn = jnp.minimum(g + 1, n_g - 1)
                drain_group(s)
                issue_group(gn, 1 - s)
                pltpu.sync_copy(
                    rows_v.at[s],
                    out_hbm.at[pl.ds((ch0 + g * _K) * _CH, _K * _CH)])

        drain_group(0)

    return gather_kernel(table, idx2d)


def _make_pe(seq_len, d):
    pos = np.arange(seq_len, dtype=np.float64)[:, None]
    div = np.exp(np.arange(0, d, 2, dtype=np.float64) * (-np.log(10000.0) / d))
    pe = np.zeros((seq_len, d), np.float64)
    pe[:, 0::2] = np.sin(pos * div)
    pe[:, 1::2] = np.cos(pos * div)
    return jnp.asarray(pe, jnp.float32)


def _project(g3, w1t, w2t, pe, b2):
    bsz, seq, d = g3.shape
    bb = 64
    assert bsz % bb == 0

    def body(g_ref, w1t_ref, w2t_ref, pe_ref, b_ref, o_ref):
        c = jnp.dot(pe_ref[...], w2t_ref[...],
                    preferred_element_type=jnp.float32) + b_ref[...]
        x = g_ref[...].reshape(bb * seq, d)
        y = jnp.dot(x, w1t_ref[...], preferred_element_type=jnp.float32)
        o_ref[...] = y.reshape(bb, seq, d) + c[None, :, :]

    return pl.pallas_call(
        body,
        grid=(bsz // bb,),
        in_specs=[
            pl.BlockSpec((bb, seq, d), lambda i: (i, 0, 0)),
            pl.BlockSpec((d, d), lambda i: (0, 0)),
            pl.BlockSpec((d, d), lambda i: (0, 0)),
            pl.BlockSpec((seq, d), lambda i: (0, 0)),
            pl.BlockSpec((1, d), lambda i: (0, 0)),
        ],
        out_specs=pl.BlockSpec((bb, seq, d), lambda i: (i, 0, 0)),
        out_shape=jax.ShapeDtypeStruct((bsz, seq, d), jnp.float32),
    )(g3, w1t, w2t, pe, b2)


def kernel(X, embed_table, W, b):
    bsz, seq = X.shape
    v, d = embed_table.shape
    n = bsz * seq
    idx2d = X.reshape(n // _CH, _CH).astype(jnp.int32)
    g = _sc_gather(embed_table, idx2d)
    g3 = g.reshape(bsz, seq, d)
    pe = _make_pe(seq, d)
    return _project(g3, W[:, :d].T, W[:, d:].T, pe, b.reshape(1, d))


# speedup vs baseline: 1.2717x; 1.2717x over previous
import functools

import numpy as np
import jax
import jax.numpy as jnp
from jax import lax
from jax.experimental import pallas as pl
from jax.experimental.pallas import tpu as pltpu
from jax.experimental.pallas import tpu_sc as plsc

_NC, _NS = 2, 16
_NW = _NC * _NS
_CH = 128
_K = 4


def _sc_gather(table, idx2d):
    n_ch_total, ch = idx2d.shape
    v, d = table.shape
    assert ch == _CH
    n_rows = n_ch_total * _CH
    n_ch = n_ch_total // _NW
    n_g = n_ch // _K
    assert n_ch_total % _NW == 0 and n_ch % _K == 0 and n_g % 2 == 0

    mesh = plsc.VectorSubcoreMesh(
        core_axis_name="c", subcore_axis_name="s",
        num_cores=_NC, num_subcores=_NS)

    @functools.partial(
        pl.kernel,
        out_type=jax.ShapeDtypeStruct((n_rows, d), jnp.float32),
        mesh=mesh,
        scratch_types=[
            pltpu.VMEM((n_ch, _CH), jnp.int32),
            pltpu.VMEM((2, _K * _CH, d), jnp.float32),
            pltpu.SemaphoreType.DMA,
            pltpu.SemaphoreType.DMA,
        ],
        compiler_params=pltpu.CompilerParams(use_tc_tiling_on_sc=False),
    )
    def gather_kernel(table_hbm, idx_hbm, out_hbm, idx_v, rows_v, sem0, sem1):
        wid = lax.axis_index("s") * _NC + lax.axis_index("c")
        ch0 = wid * n_ch
        sems = (sem0, sem1)

        pltpu.sync_copy(idx_hbm.at[pl.ds(ch0, n_ch)], idx_v)

        def issue_group(g, slot):
            for k in range(_K):
                pltpu.async_copy(
                    table_hbm.at[idx_v.at[g * _K + k]],
                    rows_v.at[slot].at[pl.ds(k * _CH, _CH)],
                    sems[slot])

        def drain_group(slot):
            for k in range(_K):
                pltpu.make_async_copy(
                    table_hbm.at[idx_v.at[0]],
                    rows_v.at[slot].at[pl.ds(k * _CH, _CH)],
                    sems[slot]).wait()

        issue_group(0, 0)

        @pl.loop(0, n_g // 2)
        def _(q):
            for s in range(2):
                g = q * 2 + s
                gn = jnp.minimum(g + 1, n_g - 1)
                drain_group(s)
                issue_group(gn, 1 - s)
                pltpu.sync_copy(
                    rows_v.at[s],
                    out_hbm.at[pl.ds((ch0 + g * _K) * _CH, _K * _CH)])

        drain_group(0)

    return gather_kernel(table, idx2d)


def _make_pe(seq_len, d):
    pos = np.arange(seq_len, dtype=np.float64)[:, None]
    div = np.exp(np.arange(0, d, 2, dtype=np.float64) * (-np.log(10000.0) / d))
    pe = np.zeros((seq_len, d), np.float64)
    pe[:, 0::2] = np.sin(pos * div)
    pe[:, 1::2] = np.cos(pos * div)
    return jnp.asarray(pe, jnp.float32)


def _project_t(g3, w1t, w2t, pe, b2):
    seq, bsz, d = g3.shape
    ll = 4
    assert seq % ll == 0

    def body(g_ref, w1t_ref, w2t_ref, pe_ref, b_ref, o_ref):
        pid = pl.program_id(0)
        pe_blk = pe_ref[pl.ds(pid * ll, ll), :]
        c = jnp.dot(pe_blk, w2t_ref[...],
                    preferred_element_type=jnp.float32) + b_ref[...]
        x = g_ref[...]
        for l in range(ll):
            y = lax.dot_general(
                w1t_ref[...], x[l],
                dimension_numbers=(((0,), (1,)), ((), ())),
                preferred_element_type=jnp.float32)
            o_ref[l, :, :] = y + c[l, :, None]

    return pl.pallas_call(
        body,
        grid=(seq // ll,),
        in_specs=[
            pl.BlockSpec((ll, bsz, d), lambda i: (i, 0, 0)),
            pl.BlockSpec((d, d), lambda i: (0, 0)),
            pl.BlockSpec((d, d), lambda i: (0, 0)),
            pl.BlockSpec((seq, d), lambda i: (0, 0)),
            pl.BlockSpec((1, d), lambda i: (0, 0)),
        ],
        out_specs=pl.BlockSpec((ll, d, bsz), lambda i: (i, 0, 0)),
        out_shape=jax.ShapeDtypeStruct((seq, d, bsz), jnp.float32),
    )(g3, w1t, w2t, pe, b2)


def kernel(X, embed_table, W, b):
    bsz, seq = X.shape
    v, d = embed_table.shape
    n = bsz * seq
    idx2d = X.T.reshape(n // _CH, _CH).astype(jnp.int32)
    g = _sc_gather(embed_table, idx2d)
    g3 = g.reshape(seq, bsz, d)
    pe = _make_pe(seq, d)
    out_t = _project_t(g3, W[:, :d].T, W[:, d:].T, pe, b.reshape(1, d))
    return jnp.transpose(out_t, (2, 0, 1))


# speedup vs baseline: 1.4572x; 1.1458x over previous
import functools

import numpy as np
import jax
import jax.numpy as jnp
from jax import lax
from jax.experimental import pallas as pl
from jax.experimental.pallas import tpu as pltpu
from jax.experimental.pallas import tpu_sc as plsc

_NC, _NS = 2, 16
_NW = _NC * _NS
_CH = 128
_K = 2


def _sc_gather(table, idx2d):
    n_ch_total, ch = idx2d.shape
    v, d = table.shape
    assert ch == _CH
    n_rows = n_ch_total * _CH
    n_ch = n_ch_total // _NW
    n_g = n_ch // _K
    assert n_ch_total % _NW == 0 and n_ch % _K == 0 and n_g % 2 == 0

    mesh = plsc.VectorSubcoreMesh(
        core_axis_name="c", subcore_axis_name="s",
        num_cores=_NC, num_subcores=_NS)

    @functools.partial(
        pl.kernel,
        out_type=jax.ShapeDtypeStruct((n_rows, d), jnp.float32),
        mesh=mesh,
        scratch_types=[
            pltpu.VMEM((n_ch, _CH), jnp.int32),
            pltpu.VMEM((2, _K * _CH, d), jnp.float32),
            pltpu.SemaphoreType.DMA,
            pltpu.SemaphoreType.DMA,
        ],
        compiler_params=pltpu.CompilerParams(use_tc_tiling_on_sc=False),
    )
    def gather_kernel(table_hbm, idx_hbm, out_hbm, idx_v, rows_v, sem0, sem1):
        wid = lax.axis_index("s") * _NC + lax.axis_index("c")
        ch0 = wid * n_ch
        sems = (sem0, sem1)

        pltpu.sync_copy(idx_hbm.at[pl.ds(ch0, n_ch)], idx_v)

        def issue_group(g, slot):
            for k in range(_K):
                pltpu.async_copy(
                    table_hbm.at[idx_v.at[g * _K + k]],
                    rows_v.at[slot].at[pl.ds(k * _CH, _CH)],
                    sems[slot])

        def drain_group(slot):
            for k in range(_K):
                pltpu.make_async_copy(
                    table_hbm.at[idx_v.at[0]],
                    rows_v.at[slot].at[pl.ds(k * _CH, _CH)],
                    sems[slot]).wait()

        issue_group(0, 0)

        @pl.loop(0, n_g // 2)
        def _(q):
            for s in range(2):
                g = q * 2 + s
                gn = jnp.minimum(g + 1, n_g - 1)
                drain_group(s)
                issue_group(gn, 1 - s)
                pltpu.sync_copy(
                    rows_v.at[s],
                    out_hbm.at[pl.ds((ch0 + g * _K) * _CH, _K * _CH)])

        drain_group(0)

    return gather_kernel(table, idx2d)


def _make_pe(seq_len, d):
    pos = np.arange(seq_len, dtype=np.float64)[:, None]
    div = np.exp(np.arange(0, d, 2, dtype=np.float64) * (-np.log(10000.0) / d))
    pe = np.zeros((seq_len, d), np.float64)
    pe[:, 0::2] = np.sin(pos * div)
    pe[:, 1::2] = np.cos(pos * div)
    return jnp.asarray(pe, jnp.float32)


def _project_t(g3, par, wbd, w2t, pe, b2):
    seq, bsz, d2 = g3.shape
    d = d2 // 2
    ll = 4
    assert seq % ll == 0

    def body(g_ref, p_ref, wbd_ref, w2t_ref, pe_ref, b_ref, o_ref):
        pid = pl.program_id(0)
        pe_blk = pe_ref[pl.ds(pid * ll, ll), :]
        c = jnp.dot(pe_blk, w2t_ref[...],
                    preferred_element_type=jnp.float32) + b_ref[...]
        x = g_ref[...]
        p = p_ref[0]
        for l in range(ll):
            y = lax.dot_general(
                wbd_ref[...], x[l],
                dimension_numbers=(((0,), (1,)), ((), ())),
                preferred_element_type=jnp.float32)
            sel = jnp.where(p[l][None, :] != 0, y[d:d2, :], y[0:d, :])
            o_ref[l, :, :] = sel + c[l, :, None]

    return pl.pallas_call(
        body,
        grid=(seq // ll,),
        in_specs=[
            pl.BlockSpec((ll, bsz, d2), lambda i: (i, 0, 0)),
            pl.BlockSpec((1, ll, bsz), lambda i: (i, 0, 0)),
            pl.BlockSpec((d2, d2), lambda i: (0, 0)),
            pl.BlockSpec((d, d), lambda i: (0, 0)),
            pl.BlockSpec((seq, d), lambda i: (0, 0)),
            pl.BlockSpec((1, d), lambda i: (0, 0)),
        ],
        out_specs=pl.BlockSpec((ll, d, bsz), lambda i: (i, 0, 0)),
        out_shape=jax.ShapeDtypeStruct((seq, d, bsz), jnp.float32),
    )(g3, par.reshape(seq // ll, ll, bsz), wbd, w2t, pe, b2)


def kernel(X, embed_table, W, b):
    bsz, seq = X.shape
    v, d = embed_table.shape
    n = bsz * seq
    xt = X.T.astype(jnp.int32)
    idx2d = (xt >> 1).reshape(n // _CH, _CH)
    par = (xt & 1)
    tbl2 = embed_table.reshape(v // 2, 2 * d)
    g = _sc_gather(tbl2, idx2d)
    g3 = g.reshape(seq, bsz, 2 * d)
    pe = _make_pe(seq, d)
    w1t = W[:, :d].T
    wbd = jnp.kron(jnp.eye(2, dtype=W.dtype), w1t)
    out_t = _project_t(g3, par, wbd, W[:, d:].T, pe, b.reshape(1, d))
    return jnp.transpose(out_t, (2, 0, 1))
